# baseline (speedup 1.0000x reference)
import functools

import jax
import jax.numpy as jnp
from jax import lax
from jax.experimental import pallas as pl
from jax.experimental.pallas import tpu as pltpu
from jax.experimental.pallas import tpu_sc as plsc

N = 10000
E = 320000
D = 128
AVG_NUM_NEIGHBORS = 32.0

NBLK = 1000
EBLK = 2048

_NC, _NS = 2, 16
NW = _NC * _NS
CH = 128
KCH = 80
EPW = KCH * CH
E_PAD = NW * EPW
ROWS2D = E_PAD // CH
N_PAD = 10240
RPS = N_PAD // _NS
ZROWS = 128

_sc_mesh = plsc.VectorSubcoreMesh(core_axis_name="c", subcore_axis_name="s")


def _precompute_body(nf_ref, wsc_ref, wup_ref, w1s_ref, w1r_ref,
                     ps_ref, pr_ref, u_ref):
    nf = nf_ref[...]
    ns = jnp.dot(nf, wsc_ref[...], preferred_element_type=jnp.float32)
    ps_ref[...] = jnp.dot(ns, w1s_ref[...], preferred_element_type=jnp.float32)
    pr_ref[...] = jnp.dot(ns, w1r_ref[...], preferred_element_type=jnp.float32)
    u_ref[...] = jnp.dot(nf, wup_ref[...], preferred_element_type=jnp.float32)


def _precompute(node_feats, W_scalar, W_up, W1s, W1r):
    blk = pl.BlockSpec((NBLK, D), lambda i: (i, 0))
    wblk = pl.BlockSpec((D, D), lambda i: (0, 0))
    return pl.pallas_call(
        _precompute_body,
        grid=(N // NBLK,),
        in_specs=[blk, wblk, wblk, wblk, wblk],
        out_specs=[blk, blk, blk],
        out_shape=[jax.ShapeDtypeStruct((N, D), jnp.float32)] * 3,
    )(node_feats, W_scalar, W_up, W1s, W1r)


TG = KCH // 2


@functools.partial(
    pl.kernel,
    out_type=[jax.ShapeDtypeStruct((E_PAD, D), jnp.float32),
              jax.ShapeDtypeStruct((E_PAD, D), jnp.float32)],
    mesh=_sc_mesh,
    scratch_types=[
        pltpu.VMEM((KCH, CH), jnp.int32),
        pltpu.VMEM((KCH, CH), jnp.int32),
        pltpu.VMEM((CH, D), jnp.float32),
        pltpu.VMEM((CH, D), jnp.float32),
        pltpu.VMEM((CH, D), jnp.float32),
        pltpu.VMEM((CH, D), jnp.float32),
        pltpu.SemaphoreType.DMA,
        pltpu.SemaphoreType.DMA,
        pltpu.SemaphoreType.DMA,
        pltpu.SemaphoreType.DMA,
    ],
)
def _gather_sc(ps_hbm, pr_hbm, s2d_hbm, r2d_hbm, gs_hbm, gr_hbm,
               sidx, ridx, rs0, rr0, rs1, rr1,
               sem_g0, sem_g1, sem_wb0, sem_wb1):
    wid = lax.axis_index("s") * _NC + lax.axis_index("c")
    krow = wid * KCH
    pltpu.sync_copy(s2d_hbm.at[pl.ds(krow, KCH)], sidx)
    pltpu.sync_copy(r2d_hbm.at[pl.ds(krow, KCH)], ridx)
    ebase = wid * EPW

    slots = ((rs0, rr0, sem_g0, sem_wb0), (rs1, rr1, sem_g1, sem_wb1))

    def fire_g(c, slot):
        rs, rr, sg, _ = slots[slot]
        pltpu.async_copy(ps_hbm.at[sidx.at[c]], rs, sg)
        pltpu.async_copy(pr_hbm.at[ridx.at[c]], rr, sg)

    def wait_g(slot):
        rs, rr, sg, _ = slots[slot]
        pltpu.make_async_copy(ps_hbm.at[sidx.at[0]], rs, sg).wait()
        pltpu.make_async_copy(pr_hbm.at[ridx.at[0]], rr, sg).wait()

    def wait_wb(slot):
        rs, rr, _, swb = slots[slot]
        pltpu.make_async_copy(rs, gs_hbm.at[pl.ds(0, CH)], swb).wait()
        pltpu.make_async_copy(rr, gr_hbm.at[pl.ds(0, CH)], swb).wait()

    fire_g(0, 0)

    def group(t, _):
        for j in (0, 1):
            c = 2 * t + j
            rs, rr, sg, swb = slots[j]
            wait_g(j)
            if j == 0:
                @pl.when(t > 0)
                def _():
                    wait_wb(1)
                fire_g(c + 1, 1)
            else:
                wait_wb(0)

                @pl.when(t < TG - 1)
                def _():
                    fire_g(c + 1, 0)

            pltpu.async_copy(rs, gs_hbm.at[pl.ds(ebase + c * CH, CH)], swb)
            pltpu.async_copy(rr, gr_hbm.at[pl.ds(ebase + c * CH, CH)], swb)
        return 0

    lax.fori_loop(0, TG, group, 0)
    wait_wb(1)


CH_S = 64
KCS = EPW // CH_S
NSEG = 4
SEG = KCS // NSEG
TG_S = SEG // 2
ROWS2DS = E_PAD // CH_S


@functools.partial(
    pl.kernel,
    out_type=jax.ShapeDtypeStruct((_NC * N_PAD, D), jnp.float32),
    mesh=_sc_mesh,
    scratch_types=[
        pltpu.VMEM((SEG, CH_S), jnp.int32),
        pltpu.VMEM((SEG, CH_S), jnp.int32),
        pltpu.VMEM((CH_S, D), jnp.float32),
        pltpu.VMEM((CH_S, D), jnp.float32),
        pltpu.VMEM((CH_S, D), jnp.float32),
        pltpu.VMEM((CH_S, D), jnp.float32),
        pltpu.VMEM_SHARED((N_PAD, D), jnp.float32),
        pltpu.SemaphoreType.DMA,
        pltpu.SemaphoreType.DMA,
        pltpu.SemaphoreType.DMA,
        pltpu.SemaphoreType.DMA,
    ],
)
def _scatter_sc(w_hbm, u_hbm, s2d_hbm, r2d_hbm, out_hbm,
                sidx, ridx, w0, u0, w1, u1, acc,
                sem_l0, sem_l1, sem_sc0, sem_sc1):
    cid = lax.axis_index("c")
    sid = lax.axis_index("s")
    wid = sid * _NC + cid

    zero16 = jnp.zeros((16,), jnp.float32)

    def zrow(i, _):
        for q in range(D // 16):
            w0[i, pl.ds(q * 16, 16)] = zero16
        return 0

    lax.fori_loop(0, CH_S, zrow, 0)

    def zcopy(t, _):
        pltpu.sync_copy(w0, acc.at[pl.ds(sid * RPS + t * CH_S, CH_S)])
        return 0

    lax.fori_loop(0, RPS // CH_S, zcopy, 0)
    plsc.subcore_barrier()

    slots = ((w0, u0, sem_l0, sem_sc0), (w1, u1, sem_l1, sem_sc1))

    def half(h, _):
        krow = wid * KCS + h * SEG
        pltpu.sync_copy(s2d_hbm.at[pl.ds(krow, SEG)], sidx)
        pltpu.sync_copy(r2d_hbm.at[pl.ds(krow, SEG)], ridx)
        ebase = wid * EPW + h * SEG * CH_S

        def fire_l(c, slot):
            w, u, sl_, _ = slots[slot]
            pltpu.async_copy(w_hbm.at[pl.ds(ebase + c * CH_S, CH_S)], w, sl_)
            pltpu.async_copy(u_hbm.at[sidx.at[c]], u, sl_)

        def wait_l(slot):
            w, u, sl_, _ = slots[slot]
            pltpu.make_async_copy(w_hbm.at[pl.ds(0, CH_S)], w, sl_).wait()
            pltpu.make_async_copy(u_hbm.at[sidx.at[0]], u, sl_).wait()

        def wait_sc(slot):
            w, _, _, ssc = slots[slot]
            pltpu.make_async_copy(w, acc.at[ridx.at[0]], ssc).wait()

        fire_l(0, 0)

        def group(t, _):
            for j in (0, 1):
                c = 2 * t + j
                w, u, _, ssc = slots[j]
                wait_l(j)
                if j == 0:
                    @pl.when(t > 0)
                    def _():
                        wait_sc(1)

                    fire_l(c + 1, 1)
                else:
                    wait_sc(0)

                    @pl.when(t < TG_S - 1)
                    def _():
                        fire_l(c + 1, 0)

                def mrow(i, _):
                    for q in range(D // 16):
                        sl = pl.ds(q * 16, 16)
                        w[i, sl] = w[i, sl] * u[i, sl]
                    return 0

                lax.fori_loop(0, CH_S, mrow, 0)
                pltpu.async_copy(w, acc.at[ridx.at[c]], ssc, add=True)
            return 0

        lax.fori_loop(0, TG_S, group, 0)
        wait_sc(1)
        return 0

    lax.fori_loop(0, NSEG, half, 0)
    plsc.subcore_barrier()

    def wout(t, _):
        rb = sid * RPS + t * CH_S
        pltpu.sync_copy(acc.at[pl.ds(rb, CH_S)],
                        out_hbm.at[pl.ds(cid * N_PAD + rb, CH_S)])
        return 0

    lax.fori_loop(0, RPS // CH_S, wout, 0)


def _mlp_body(gs_ref, gr_ref, ef_ref, ea_ref, w1e_ref, b1_ref, w2_ref, b2_ref,
              w3_ref, w_ref):
    g = gs_ref[...] + gr_ref[...]
    et = jnp.dot(ef_ref[...], w1e_ref[...], preferred_element_type=jnp.float32)
    h = g + et + b1_ref[...]
    h = h * jax.nn.sigmoid(h)
    h = jnp.dot(h, w2_ref[...], preferred_element_type=jnp.float32) + b2_ref[...]
    h = h * jax.nn.sigmoid(h)
    tpw = jnp.dot(h, w3_ref[...], preferred_element_type=jnp.float32)
    w_ref[...] = tpw * ea_ref[...]


def _mlp(Gs, Gr, ef_ext, edge_attrs, W1e, b1, W2, b2, W3):
    eblk = pl.BlockSpec((EBLK, D), lambda i: (i, 0))
    return pl.pallas_call(
        _mlp_body,
        grid=(E_PAD // EBLK,),
        in_specs=[
            eblk,
            eblk,
            pl.BlockSpec((EBLK, 17), lambda i: (i, 0)),
            pl.BlockSpec((EBLK, 1), lambda i: (i, 0)),
            pl.BlockSpec((17, D), lambda i: (0, 0)),
            pl.BlockSpec((1, D), lambda i: (0, 0)),
            pl.BlockSpec((D, D), lambda i: (0, 0)),
            pl.BlockSpec((1, D), lambda i: (0, 0)),
            pl.BlockSpec((D, D), lambda i: (0, 0)),
        ],
        out_specs=eblk,
        out_shape=jax.ShapeDtypeStruct((E_PAD, D), jnp.float32),
    )(Gs, Gr, ef_ext, edge_attrs, W1e, b1, W2, b2, W3)


def _final_body(m_ref, wout_ref, out_ref):
    m = m_ref[0] + m_ref[1]
    out_ref[...] = jnp.dot(m, wout_ref[...],
                           preferred_element_type=jnp.float32) * (1.0 / AVG_NUM_NEIGHBORS)


def _final(message_parts, W_out):
    return pl.pallas_call(
        _final_body,
        grid=(N // NBLK,),
        in_specs=[
            pl.BlockSpec((2, NBLK, D), lambda i: (0, i, 0)),
            pl.BlockSpec((D, D), lambda i: (0, 0)),
        ],
        out_specs=pl.BlockSpec((NBLK, D), lambda i: (i, 0)),
        out_shape=jax.ShapeDtypeStruct((N, D), jnp.float32),
    )(message_parts, W_out)


def kernel(node_feats, edge_attrs, edge_feats, lengths, edge_index,
           W_scalar, W_up, W1, b1, W2, b2, W3, W_out):
    sender = edge_index[0].astype(jnp.int32)
    receiver = edge_index[1].astype(jnp.int32)

    W1s = W1[:D]
    W1r = W1[D:2 * D]
    W1e = W1[2 * D:]

    P_s, P_r, U = _precompute(node_feats, W_scalar, W_up, W1s, W1r)

    pad = E_PAD - E
    s2d = jnp.pad(sender, (0, pad)).reshape(ROWS2D, CH)
    r2d = jnp.pad(receiver, (0, pad)).reshape(ROWS2D, CH)
    ef_ext = jnp.pad(jnp.concatenate([edge_feats, lengths], axis=1),
                     ((0, pad), (0, 0)))
    ea_pad = jnp.pad(edge_attrs, ((0, pad), (0, 0)))

    s2ds = s2d.reshape(ROWS2DS, CH_S)
    r2ds = r2d.reshape(ROWS2DS, CH_S)

    Gs, Gr = _gather_sc(P_s, P_r, s2d, r2d)

    w = _mlp(Gs, Gr, ef_ext, ea_pad, W1e,
             b1.reshape(1, D), W2, b2.reshape(1, D), W3)

    message_parts = _scatter_sc(w, U, s2ds, r2ds).reshape(_NC, N_PAD, D)[:, :N, :]

    out = _final(message_parts, W_out)
    return out.reshape(N, D, 1)


# speedup vs baseline: 1.1289x; 1.1289x over previous
import functools

import jax
import jax.numpy as jnp
from jax import lax
from jax.experimental import pallas as pl
from jax.experimental.pallas import tpu as pltpu
from jax.experimental.pallas import tpu_sc as plsc

N = 10000
E = 320000
D = 128
AVG_NUM_NEIGHBORS = 32.0

NBLK = 1000
EBLK = 2048

_NC, _NS = 2, 16
NW = _NC * _NS
CH = 128
KCH = 80
EPW = KCH * CH
E_PAD = NW * EPW
ROWS2D = E_PAD // CH
N_PAD = 10240
RPS = N_PAD // _NS
ZROWS = 128

_sc_mesh = plsc.VectorSubcoreMesh(core_axis_name="c", subcore_axis_name="s")


def _precompute_body(nf_ref, wsc_ref, wup_ref, w1s_ref, w1r_ref,
                     ps_ref, pr_ref, u_ref):
    nf = nf_ref[...]
    ns = jnp.dot(nf, wsc_ref[...], preferred_element_type=jnp.float32)
    ps_ref[...] = jnp.dot(ns, w1s_ref[...], preferred_element_type=jnp.float32)
    pr_ref[...] = jnp.dot(ns, w1r_ref[...], preferred_element_type=jnp.float32)
    u_ref[...] = jnp.dot(nf, wup_ref[...], preferred_element_type=jnp.float32)


def _precompute(node_feats, W_scalar, W_up, W1s, W1r):
    blk = pl.BlockSpec((NBLK, D), lambda i: (i, 0))
    wblk = pl.BlockSpec((D, D), lambda i: (0, 0))
    return pl.pallas_call(
        _precompute_body,
        grid=(N // NBLK,),
        in_specs=[blk, wblk, wblk, wblk, wblk],
        out_specs=[blk, blk, blk],
        out_shape=[jax.ShapeDtypeStruct((N, D), jnp.float32)] * 3,
    )(node_feats, W_scalar, W_up, W1s, W1r)


CHG = 64
KCG = EPW // CHG
TGG = KCG // 4


@functools.partial(
    pl.kernel,
    out_type=jax.ShapeDtypeStruct((E_PAD, D), jnp.float32),
    mesh=_sc_mesh,
    scratch_types=[
        pltpu.VMEM((KCG, CHG), jnp.int32),
        pltpu.VMEM((KCG, CHG), jnp.int32),
    ] + [pltpu.VMEM((CHG, D), jnp.float32)] * 8
      + [pltpu.SemaphoreType.DMA] * 8,
)
def _gather_sc(ps_hbm, pr_hbm, s2d_hbm, r2d_hbm, g_hbm,
               sidx, ridx, rs0, rr0, rs1, rr1, rs2, rr2, rs3, rr3,
               sg0, sg1, sg2, sg3, sw0, sw1, sw2, sw3):
    wid = lax.axis_index("s") * _NC + lax.axis_index("c")
    krow = wid * KCG
    pltpu.sync_copy(s2d_hbm.at[pl.ds(krow, KCG)], sidx)
    pltpu.sync_copy(r2d_hbm.at[pl.ds(krow, KCG)], ridx)
    ebase = wid * EPW

    slots = ((rs0, rr0, sg0, sw0), (rs1, rr1, sg1, sw1),
             (rs2, rr2, sg2, sw2), (rs3, rr3, sg3, sw3))

    def fire_g(c, slot):
        rs, rr, sg, _ = slots[slot]
        pltpu.async_copy(ps_hbm.at[sidx.at[c]], rs, sg)
        pltpu.async_copy(pr_hbm.at[ridx.at[c]], rr, sg)

    def wait_g(slot):
        rs, rr, sg, _ = slots[slot]
        pltpu.make_async_copy(ps_hbm.at[sidx.at[0]], rs, sg).wait()
        pltpu.make_async_copy(pr_hbm.at[ridx.at[0]], rr, sg).wait()

    def wait_wb(slot):
        rs, _, _, swb = slots[slot]
        pltpu.make_async_copy(rs, g_hbm.at[pl.ds(0, CHG)], swb).wait()

    fire_g(0, 0)
    fire_g(1, 1)
    fire_g(2, 2)

    def group(t, _):
        for j in range(4):
            c = 4 * t + j
            rs, rr, _, swb = slots[j]
            nslot = (j + 3) % 4
            wait_g(j)
            if j == 0:
                @pl.when(t > 0)
                def _():
                    wait_wb(nslot)

                fire_g(c + 3, nslot)
            else:
                wait_wb(nslot)

                @pl.when(t < TGG - 1)
                def _():
                    fire_g(c + 3, nslot)

            def add_row(i, _):
                for q in range(D // 16):
                    sl = pl.ds(q * 16, 16)
                    rs[i, sl] = rs[i, sl] + rr[i, sl]
                return 0

            lax.fori_loop(0, CHG, add_row, 0)
            pltpu.async_copy(rs, g_hbm.at[pl.ds(ebase + c * CHG, CHG)], swb)
        return 0

    lax.fori_loop(0, TGG, group, 0)
    wait_wb(3)


CH_S = 64
KCS = EPW // CH_S
NSEG = 4
SEG = KCS // NSEG
TG_S = SEG // 2
ROWS2DS = E_PAD // CH_S


@functools.partial(
    pl.kernel,
    out_type=jax.ShapeDtypeStruct((_NC * N_PAD, D), jnp.float32),
    mesh=_sc_mesh,
    scratch_types=[
        pltpu.VMEM((SEG, CH_S), jnp.int32),
        pltpu.VMEM((SEG, CH_S), jnp.int32),
        pltpu.VMEM((CH_S, D), jnp.float32),
        pltpu.VMEM((CH_S, D), jnp.float32),
        pltpu.VMEM((CH_S, D), jnp.float32),
        pltpu.VMEM((CH_S, D), jnp.float32),
        pltpu.VMEM_SHARED((N_PAD, D), jnp.float32),
        pltpu.SemaphoreType.DMA,
        pltpu.SemaphoreType.DMA,
        pltpu.SemaphoreType.DMA,
        pltpu.SemaphoreType.DMA,
    ],
)
def _scatter_sc(w_hbm, u_hbm, s2d_hbm, r2d_hbm, out_hbm,
                sidx, ridx, w0, u0, w1, u1, acc,
                sem_l0, sem_l1, sem_sc0, sem_sc1):
    cid = lax.axis_index("c")
    sid = lax.axis_index("s")
    wid = sid * _NC + cid

    zero16 = jnp.zeros((16,), jnp.float32)

    def zrow(i, _):
        for q in range(D // 16):
            w0[i, pl.ds(q * 16, 16)] = zero16
        return 0

    lax.fori_loop(0, CH_S, zrow, 0)

    def zcopy(t, _):
        pltpu.sync_copy(w0, acc.at[pl.ds(sid * RPS + t * CH_S, CH_S)])
        return 0

    lax.fori_loop(0, RPS // CH_S, zcopy, 0)
    plsc.subcore_barrier()

    slots = ((w0, u0, sem_l0, sem_sc0), (w1, u1, sem_l1, sem_sc1))

    def half(h, _):
        krow = wid * KCS + h * SEG
        pltpu.sync_copy(s2d_hbm.at[pl.ds(krow, SEG)], sidx)
        pltpu.sync_copy(r2d_hbm.at[pl.ds(krow, SEG)], ridx)
        ebase = wid * EPW + h * SEG * CH_S

        def fire_l(c, slot):
            w, u, sl_, _ = slots[slot]
            pltpu.async_copy(w_hbm.at[pl.ds(ebase + c * CH_S, CH_S)], w, sl_)
            pltpu.async_copy(u_hbm.at[sidx.at[c]], u, sl_)

        def wait_l(slot):
            w, u, sl_, _ = slots[slot]
            pltpu.make_async_copy(w_hbm.at[pl.ds(0, CH_S)], w, sl_).wait()
            pltpu.make_async_copy(u_hbm.at[sidx.at[0]], u, sl_).wait()

        def wait_sc(slot):
            w, _, _, ssc = slots[slot]
            pltpu.make_async_copy(w, acc.at[ridx.at[0]], ssc).wait()

        fire_l(0, 0)

        def group(t, _):
            for j in (0, 1):
                c = 2 * t + j
                w, u, _, ssc = slots[j]
                wait_l(j)
                if j == 0:
                    @pl.when(t > 0)
                    def _():
                        wait_sc(1)

                    fire_l(c + 1, 1)
                else:
                    wait_sc(0)

                    @pl.when(t < TG_S - 1)
                    def _():
                        fire_l(c + 1, 0)

                def mrow(i, _):
                    for q in range(D // 16):
                        sl = pl.ds(q * 16, 16)
                        w[i, sl] = w[i, sl] * u[i, sl]
                    return 0

                lax.fori_loop(0, CH_S, mrow, 0)
                pltpu.async_copy(w, acc.at[ridx.at[c]], ssc, add=True)
            return 0

        lax.fori_loop(0, TG_S, group, 0)
        wait_sc(1)
        return 0

    lax.fori_loop(0, NSEG, half, 0)
    plsc.subcore_barrier()

    def wout(t, _):
        rb = sid * RPS + t * CH_S
        pltpu.sync_copy(acc.at[pl.ds(rb, CH_S)],
                        out_hbm.at[pl.ds(cid * N_PAD + rb, CH_S)])
        return 0

    lax.fori_loop(0, RPS // CH_S, wout, 0)


def _mlp_body(g_ref, ef_ref, ea_ref, w1e_ref, b1_ref, w2_ref, b2_ref,
              w3_ref, w_ref):
    g = g_ref[...]
    et = jnp.dot(ef_ref[...], w1e_ref[...], preferred_element_type=jnp.float32)
    h = g + et + b1_ref[...]
    h = h * jax.nn.sigmoid(h)
    h = jnp.dot(h, w2_ref[...], preferred_element_type=jnp.float32) + b2_ref[...]
    h = h * jax.nn.sigmoid(h)
    tpw = jnp.dot(h, w3_ref[...], preferred_element_type=jnp.float32)
    w_ref[...] = tpw * ea_ref[...]


def _mlp(G, ef_ext, edge_attrs, W1e, b1, W2, b2, W3):
    eblk = pl.BlockSpec((EBLK, D), lambda i: (i, 0))
    return pl.pallas_call(
        _mlp_body,
        grid=(E_PAD // EBLK,),
        in_specs=[
            eblk,
            pl.BlockSpec((EBLK, 17), lambda i: (i, 0)),
            pl.BlockSpec((EBLK, 1), lambda i: (i, 0)),
            pl.BlockSpec((17, D), lambda i: (0, 0)),
            pl.BlockSpec((1, D), lambda i: (0, 0)),
            pl.BlockSpec((D, D), lambda i: (0, 0)),
            pl.BlockSpec((1, D), lambda i: (0, 0)),
            pl.BlockSpec((D, D), lambda i: (0, 0)),
        ],
        out_specs=eblk,
        out_shape=jax.ShapeDtypeStruct((E_PAD, D), jnp.float32),
    )(G, ef_ext, edge_attrs, W1e, b1, W2, b2, W3)


def _final_body(m_ref, wout_ref, out_ref):
    m = m_ref[0] + m_ref[1]
    out_ref[...] = jnp.dot(m, wout_ref[...],
                           preferred_element_type=jnp.float32) * (1.0 / AVG_NUM_NEIGHBORS)


def _final(message_parts, W_out):
    return pl.pallas_call(
        _final_body,
        grid=(N // NBLK,),
        in_specs=[
            pl.BlockSpec((2, NBLK, D), lambda i: (0, i, 0)),
            pl.BlockSpec((D, D), lambda i: (0, 0)),
        ],
        out_specs=pl.BlockSpec((NBLK, D), lambda i: (i, 0)),
        out_shape=jax.ShapeDtypeStruct((N, D), jnp.float32),
    )(message_parts, W_out)


def kernel(node_feats, edge_attrs, edge_feats, lengths, edge_index,
           W_scalar, W_up, W1, b1, W2, b2, W3, W_out):
    sender = edge_index[0].astype(jnp.int32)
    receiver = edge_index[1].astype(jnp.int32)

    W1s = W1[:D]
    W1r = W1[D:2 * D]
    W1e = W1[2 * D:]

    P_s, P_r, U = _precompute(node_feats, W_scalar, W_up, W1s, W1r)

    pad = E_PAD - E
    s2d = jnp.pad(sender, (0, pad)).reshape(ROWS2DS, CH_S)
    r2d = jnp.pad(receiver, (0, pad)).reshape(ROWS2DS, CH_S)
    ef_ext = jnp.pad(jnp.concatenate([edge_feats, lengths], axis=1),
                     ((0, pad), (0, 0)))
    ea_pad = jnp.pad(edge_attrs, ((0, pad), (0, 0)))

    G = _gather_sc(P_s, P_r, s2d, r2d)

    w = _mlp(G, ef_ext, ea_pad, W1e,
             b1.reshape(1, D), W2, b2.reshape(1, D), W3)

    message_parts = _scatter_sc(w, U, s2d, r2d).reshape(_NC, N_PAD, D)[:, :N, :]

    out = _final(message_parts, W_out)
    return out.reshape(N, D, 1)


# speedup vs baseline: 1.2612x; 1.1172x over previous
import functools

import jax
import jax.numpy as jnp
from jax import lax
from jax.experimental import pallas as pl
from jax.experimental.pallas import tpu as pltpu
from jax.experimental.pallas import tpu_sc as plsc

N = 10000
E = 320000
D = 128
AVG_NUM_NEIGHBORS = 32.0

NBLK = 1000
EBLK = 2048

_NC, _NS = 2, 16
NW = _NC * _NS
CH = 128
KCH = 80
EPW = KCH * CH
E_PAD = NW * EPW
ROWS2D = E_PAD // CH
N_PAD = 10240
RPS = N_PAD // _NS
ZROWS = 128

_sc_mesh = plsc.VectorSubcoreMesh(core_axis_name="c", subcore_axis_name="s")


def _precompute_body(nf_ref, wsc_ref, wup_ref, w1s_ref, w1r_ref,
                     ps_ref, pr_ref, u_ref):
    nf = nf_ref[...]
    ns = jnp.dot(nf, wsc_ref[...], preferred_element_type=jnp.float32)
    ps_ref[...] = jnp.dot(ns, w1s_ref[...], preferred_element_type=jnp.float32)
    pr_ref[...] = jnp.dot(ns, w1r_ref[...], preferred_element_type=jnp.float32)
    u_ref[...] = jnp.dot(nf, wup_ref[...], preferred_element_type=jnp.float32)


def _precompute(node_feats, W_scalar, W_up, W1s, W1r):
    blk = pl.BlockSpec((NBLK, D), lambda i: (i, 0))
    wblk = pl.BlockSpec((D, D), lambda i: (0, 0))
    return pl.pallas_call(
        _precompute_body,
        grid=(N // NBLK,),
        in_specs=[blk, wblk, wblk, wblk, wblk],
        out_specs=[blk, blk, blk],
        out_shape=[jax.ShapeDtypeStruct((N, D), jnp.float32)] * 3,
    )(node_feats, W_scalar, W_up, W1s, W1r)


NSEC = 2
E_SEC = E_PAD // NSEC
EPW_S = EPW // NSEC
CHG = 64
KCG = EPW_S // CHG
TGG = KCG // 4


@functools.partial(
    pl.kernel,
    out_type=jax.ShapeDtypeStruct((E_SEC, D), jnp.float32),
    mesh=_sc_mesh,
    scratch_types=[
        pltpu.VMEM((KCG, CHG), jnp.int32),
        pltpu.VMEM((KCG, CHG), jnp.int32),
    ] + [pltpu.VMEM((CHG, D), jnp.float32)] * 8
      + [pltpu.SemaphoreType.DMA] * 8,
)
def _gather_sc(ps_hbm, pr_hbm, s2d_hbm, r2d_hbm, g_hbm,
               sidx, ridx, rs0, rr0, rs1, rr1, rs2, rr2, rs3, rr3,
               sg0, sg1, sg2, sg3, sw0, sw1, sw2, sw3):
    wid = lax.axis_index("s") * _NC + lax.axis_index("c")
    krow = wid * KCG
    pltpu.sync_copy(s2d_hbm.at[pl.ds(krow, KCG)], sidx)
    pltpu.sync_copy(r2d_hbm.at[pl.ds(krow, KCG)], ridx)
    ebase = wid * EPW_S

    slots = ((rs0, rr0, sg0, sw0), (rs1, rr1, sg1, sw1),
             (rs2, rr2, sg2, sw2), (rs3, rr3, sg3, sw3))

    def fire_g(c, slot):
        rs, rr, sg, _ = slots[slot]
        pltpu.async_copy(ps_hbm.at[sidx.at[c]], rs, sg)
        pltpu.async_copy(pr_hbm.at[ridx.at[c]], rr, sg)

    def wait_g(slot):
        rs, rr, sg, _ = slots[slot]
        pltpu.make_async_copy(ps_hbm.at[sidx.at[0]], rs, sg).wait()
        pltpu.make_async_copy(pr_hbm.at[ridx.at[0]], rr, sg).wait()

    def wait_wb(slot):
        rs, _, _, swb = slots[slot]
        pltpu.make_async_copy(rs, g_hbm.at[pl.ds(0, CHG)], swb).wait()

    fire_g(0, 0)
    fire_g(1, 1)
    fire_g(2, 2)

    def group(t, _):
        for j in range(4):
            c = 4 * t + j
            rs, rr, _, swb = slots[j]
            nslot = (j + 3) % 4
            wait_g(j)
            if j == 0:
                @pl.when(t > 0)
                def _():
                    wait_wb(nslot)

                fire_g(c + 3, nslot)
            else:
                wait_wb(nslot)

                @pl.when(t < TGG - 1)
                def _():
                    fire_g(c + 3, nslot)

            def add_row(i, _):
                for q in range(D // 16):
                    sl = pl.ds(q * 16, 16)
                    rs[i, sl] = rs[i, sl] + rr[i, sl]
                return 0

            lax.fori_loop(0, CHG, add_row, 0)
            pltpu.async_copy(rs, g_hbm.at[pl.ds(ebase + c * CHG, CHG)], swb)
        return 0

    lax.fori_loop(0, TGG, group, 0)
    wait_wb(3)


CH_S = 64
KCS = EPW_S // CH_S
NSEG = 2
SEG = KCS // NSEG
TG_S = SEG // 2
ROWS2DS = E_PAD // CH_S


@functools.partial(
    pl.kernel,
    out_type=jax.ShapeDtypeStruct((_NC * N_PAD, D), jnp.float32),
    mesh=_sc_mesh,
    scratch_types=[
        pltpu.VMEM((SEG, CH_S), jnp.int32),
        pltpu.VMEM((SEG, CH_S), jnp.int32),
        pltpu.VMEM((CH_S, D), jnp.float32),
        pltpu.VMEM((CH_S, D), jnp.float32),
        pltpu.VMEM((CH_S, D), jnp.float32),
        pltpu.VMEM((CH_S, D), jnp.float32),
        pltpu.VMEM_SHARED((N_PAD, D), jnp.float32),
        pltpu.SemaphoreType.DMA,
        pltpu.SemaphoreType.DMA,
        pltpu.SemaphoreType.DMA,
        pltpu.SemaphoreType.DMA,
    ],
)
def _scatter_sc(w_hbm, u_hbm, s2d_hbm, r2d_hbm, out_hbm,
                sidx, ridx, w0, u0, w1, u1, acc,
                sem_l0, sem_l1, sem_sc0, sem_sc1):
    cid = lax.axis_index("c")
    sid = lax.axis_index("s")
    wid = sid * _NC + cid

    zero16 = jnp.zeros((16,), jnp.float32)

    def zrow(i, _):
        for q in range(D // 16):
            w0[i, pl.ds(q * 16, 16)] = zero16
        return 0

    lax.fori_loop(0, CH_S, zrow, 0)

    def zcopy(t, _):
        pltpu.sync_copy(w0, acc.at[pl.ds(sid * RPS + t * CH_S, CH_S)])
        return 0

    lax.fori_loop(0, RPS // CH_S, zcopy, 0)
    plsc.subcore_barrier()

    slots = ((w0, u0, sem_l0, sem_sc0), (w1, u1, sem_l1, sem_sc1))

    def half(h, _):
        krow = wid * KCS + h * SEG
        pltpu.sync_copy(s2d_hbm.at[pl.ds(krow, SEG)], sidx)
        pltpu.sync_copy(r2d_hbm.at[pl.ds(krow, SEG)], ridx)
        ebase = wid * EPW_S + h * SEG * CH_S

        def fire_l(c, slot):
            w, u, sl_, _ = slots[slot]
            pltpu.async_copy(w_hbm.at[pl.ds(ebase + c * CH_S, CH_S)], w, sl_)
            pltpu.async_copy(u_hbm.at[sidx.at[c]], u, sl_)

        def wait_l(slot):
            w, u, sl_, _ = slots[slot]
            pltpu.make_async_copy(w_hbm.at[pl.ds(0, CH_S)], w, sl_).wait()
            pltpu.make_async_copy(u_hbm.at[sidx.at[0]], u, sl_).wait()

        def wait_sc(slot):
            w, _, _, ssc = slots[slot]
            pltpu.make_async_copy(w, acc.at[ridx.at[0]], ssc).wait()

        fire_l(0, 0)

        def group(t, _):
            for j in (0, 1):
                c = 2 * t + j
                w, u, _, ssc = slots[j]
                wait_l(j)
                if j == 0:
                    @pl.when(t > 0)
                    def _():
                        wait_sc(1)

                    fire_l(c + 1, 1)
                else:
                    wait_sc(0)

                    @pl.when(t < TG_S - 1)
                    def _():
                        fire_l(c + 1, 0)

                def mrow(i, _):
                    for q in range(D // 16):
                        sl = pl.ds(q * 16, 16)
                        w[i, sl] = w[i, sl] * u[i, sl]
                    return 0

                lax.fori_loop(0, CH_S, mrow, 0)
                pltpu.async_copy(w, acc.at[ridx.at[c]], ssc, add=True)
            return 0

        lax.fori_loop(0, TG_S, group, 0)
        wait_sc(1)
        return 0

    lax.fori_loop(0, NSEG, half, 0)
    plsc.subcore_barrier()

    def wout(t, _):
        rb = sid * RPS + t * CH_S
        pltpu.sync_copy(acc.at[pl.ds(rb, CH_S)],
                        out_hbm.at[pl.ds(cid * N_PAD + rb, CH_S)])
        return 0

    lax.fori_loop(0, RPS // CH_S, wout, 0)


def _mlp_body(g_ref, ef_ref, ea_ref, w1e_ref, b1_ref, w2_ref, b2_ref,
              w3_ref, w_ref):
    g = g_ref[...]
    et = jnp.dot(ef_ref[...], w1e_ref[...], preferred_element_type=jnp.float32)
    h = g + et + b1_ref[...]
    h = h * jax.nn.sigmoid(h)
    h = jnp.dot(h, w2_ref[...], preferred_element_type=jnp.float32) + b2_ref[...]
    h = h * jax.nn.sigmoid(h)
    tpw = jnp.dot(h, w3_ref[...], preferred_element_type=jnp.float32)
    w_ref[...] = tpw * ea_ref[...]


def _mlp(G, ef_ext, edge_attrs, W1e, b1, W2, b2, W3):
    eblk = pl.BlockSpec((EBLK, D), lambda i: (i, 0))
    return pl.pallas_call(
        _mlp_body,
        grid=(G.shape[0] // EBLK,),
        in_specs=[
            eblk,
            pl.BlockSpec((EBLK, 17), lambda i: (i, 0)),
            pl.BlockSpec((EBLK, 1), lambda i: (i, 0)),
            pl.BlockSpec((17, D), lambda i: (0, 0)),
            pl.BlockSpec((1, D), lambda i: (0, 0)),
            pl.BlockSpec((D, D), lambda i: (0, 0)),
            pl.BlockSpec((1, D), lambda i: (0, 0)),
            pl.BlockSpec((D, D), lambda i: (0, 0)),
        ],
        out_specs=eblk,
        out_shape=jax.ShapeDtypeStruct((G.shape[0], D), jnp.float32),
    )(G, ef_ext, edge_attrs, W1e, b1, W2, b2, W3)


def _final_body(m_ref, wout_ref, out_ref):
    m = (m_ref[0] + m_ref[1]) + (m_ref[2] + m_ref[3])
    out_ref[...] = jnp.dot(m, wout_ref[...],
                           preferred_element_type=jnp.float32) * (1.0 / AVG_NUM_NEIGHBORS)


def _final(message_parts, W_out):
    return pl.pallas_call(
        _final_body,
        grid=(N // NBLK,),
        in_specs=[
            pl.BlockSpec((2 * NSEC, NBLK, D), lambda i: (0, i, 0)),
            pl.BlockSpec((D, D), lambda i: (0, 0)),
        ],
        out_specs=pl.BlockSpec((NBLK, D), lambda i: (i, 0)),
        out_shape=jax.ShapeDtypeStruct((N, D), jnp.float32),
    )(message_parts, W_out)


def kernel(node_feats, edge_attrs, edge_feats, lengths, edge_index,
           W_scalar, W_up, W1, b1, W2, b2, W3, W_out):
    sender = edge_index[0].astype(jnp.int32)
    receiver = edge_index[1].astype(jnp.int32)

    W1s = W1[:D]
    W1r = W1[D:2 * D]
    W1e = W1[2 * D:]

    P_s, P_r, U = _precompute(node_feats, W_scalar, W_up, W1s, W1r)

    pad = E_PAD - E
    s2d = jnp.pad(sender, (0, pad)).reshape(ROWS2DS, CH_S)
    r2d = jnp.pad(receiver, (0, pad)).reshape(ROWS2DS, CH_S)
    ef_ext = jnp.pad(jnp.concatenate([edge_feats, lengths], axis=1),
                     ((0, pad), (0, 0)))
    ea_pad = jnp.pad(edge_attrs, ((0, pad), (0, 0)))

    idx_rps = ROWS2DS // NSEC
    e_sec = E_SEC
    parts = []
    ws = []
    for s in range(NSEC):
        s2d_h = lax.slice_in_dim(s2d, s * idx_rps, (s + 1) * idx_rps)
        r2d_h = lax.slice_in_dim(r2d, s * idx_rps, (s + 1) * idx_rps)
        G = _gather_sc(P_s, P_r, s2d_h, r2d_h)
        w = _mlp(G, lax.slice_in_dim(ef_ext, s * e_sec, (s + 1) * e_sec),
                 lax.slice_in_dim(ea_pad, s * e_sec, (s + 1) * e_sec), W1e,
                 b1.reshape(1, D), W2, b2.reshape(1, D), W3)
        ws.append((w, s2d_h, r2d_h))
    for w, s2d_h, r2d_h in ws:
        parts.append(_scatter_sc(w, U, s2d_h, r2d_h).reshape(_NC, N_PAD, D))
    message_parts = jnp.concatenate(parts, axis=0)[:, :N, :]

    out = _final(message_parts, W_out)
    return out.reshape(N, D, 1)


# speedup vs baseline: 1.2762x; 1.0118x over previous
import functools

import jax
import jax.numpy as jnp
from jax import lax
from jax.experimental import pallas as pl
from jax.experimental.pallas import tpu as pltpu
from jax.experimental.pallas import tpu_sc as plsc

N = 10000
E = 320000
D = 128
AVG_NUM_NEIGHBORS = 32.0

NBLK = 1000
EBLK = 2048

_NC, _NS = 2, 16
NW = _NC * _NS
CH = 128
KCH = 80
EPW = KCH * CH
E_PAD = NW * EPW
ROWS2D = E_PAD // CH
N_PAD = 10240
RPS = N_PAD // _NS
ZROWS = 128

_sc_mesh = plsc.VectorSubcoreMesh(core_axis_name="c", subcore_axis_name="s")


def _precompute_body(nf_ref, wsc_ref, wup_ref, w1s_ref, w1r_ref,
                     ps_ref, pr_ref, u_ref):
    nf = nf_ref[...]
    ns = jnp.dot(nf, wsc_ref[...], preferred_element_type=jnp.float32)
    ps_ref[...] = jnp.dot(ns, w1s_ref[...], preferred_element_type=jnp.float32)
    pr_ref[...] = jnp.dot(ns, w1r_ref[...], preferred_element_type=jnp.float32)
    u_ref[...] = jnp.dot(nf, wup_ref[...], preferred_element_type=jnp.float32)


def _precompute(node_feats, W_scalar, W_up, W1s, W1r):
    blk = pl.BlockSpec((NBLK, D), lambda i: (i, 0))
    wblk = pl.BlockSpec((D, D), lambda i: (0, 0))
    return pl.pallas_call(
        _precompute_body,
        grid=(N // NBLK,),
        in_specs=[blk, wblk, wblk, wblk, wblk],
        out_specs=[blk, blk, blk],
        out_shape=[jax.ShapeDtypeStruct((N, D), jnp.float32)] * 3,
    )(node_feats, W_scalar, W_up, W1s, W1r)


NSEC = 4
E_SEC = E_PAD // NSEC
EPW_S = EPW // NSEC
CHG = 64
KCG = EPW_S // CHG
TGG = KCG // 4


@functools.partial(
    pl.kernel,
    out_type=jax.ShapeDtypeStruct((E_SEC, D), jnp.float32),
    mesh=_sc_mesh,
    scratch_types=[
        pltpu.VMEM((KCG, CHG), jnp.int32),
        pltpu.VMEM((KCG, CHG), jnp.int32),
    ] + [pltpu.VMEM((CHG, D), jnp.float32)] * 8
      + [pltpu.SemaphoreType.DMA] * 8,
)
def _gather_sc(ps_hbm, pr_hbm, s2d_hbm, r2d_hbm, g_hbm,
               sidx, ridx, rs0, rr0, rs1, rr1, rs2, rr2, rs3, rr3,
               sg0, sg1, sg2, sg3, sw0, sw1, sw2, sw3):
    wid = lax.axis_index("s") * _NC + lax.axis_index("c")
    krow = wid * KCG
    pltpu.sync_copy(s2d_hbm.at[pl.ds(krow, KCG)], sidx)
    pltpu.sync_copy(r2d_hbm.at[pl.ds(krow, KCG)], ridx)
    ebase = wid * EPW_S

    slots = ((rs0, rr0, sg0, sw0), (rs1, rr1, sg1, sw1),
             (rs2, rr2, sg2, sw2), (rs3, rr3, sg3, sw3))

    def fire_g(c, slot):
        rs, rr, sg, _ = slots[slot]
        pltpu.async_copy(ps_hbm.at[sidx.at[c]], rs, sg)
        pltpu.async_copy(pr_hbm.at[ridx.at[c]], rr, sg)

    def wait_g(slot):
        rs, rr, sg, _ = slots[slot]
        pltpu.make_async_copy(ps_hbm.at[sidx.at[0]], rs, sg).wait()
        pltpu.make_async_copy(pr_hbm.at[ridx.at[0]], rr, sg).wait()

    def wait_wb(slot):
        rs, _, _, swb = slots[slot]
        pltpu.make_async_copy(rs, g_hbm.at[pl.ds(0, CHG)], swb).wait()

    fire_g(0, 0)
    fire_g(1, 1)
    fire_g(2, 2)

    def group(t, _):
        for j in range(4):
            c = 4 * t + j
            rs, rr, _, swb = slots[j]
            nslot = (j + 3) % 4
            wait_g(j)
            if j == 0:
                @pl.when(t > 0)
                def _():
                    wait_wb(nslot)

                fire_g(c + 3, nslot)
            else:
                wait_wb(nslot)

                @pl.when(t < TGG - 1)
                def _():
                    fire_g(c + 3, nslot)

            def add_row(i, _):
                for q in range(D // 16):
                    sl = pl.ds(q * 16, 16)
                    rs[i, sl] = rs[i, sl] + rr[i, sl]
                return 0

            lax.fori_loop(0, CHG, add_row, 0)
            pltpu.async_copy(rs, g_hbm.at[pl.ds(ebase + c * CHG, CHG)], swb)
        return 0

    lax.fori_loop(0, TGG, group, 0)
    wait_wb(3)


CH_S = 64
KCS = EPW_S // CH_S
NSEG = 1
SEG = KCS // NSEG
TG_S = SEG // 2
ROWS2DS = E_PAD // CH_S


@functools.partial(
    pl.kernel,
    out_type=jax.ShapeDtypeStruct((_NC * N_PAD, D), jnp.float32),
    mesh=_sc_mesh,
    scratch_types=[
        pltpu.VMEM((SEG, CH_S), jnp.int32),
        pltpu.VMEM((SEG, CH_S), jnp.int32),
        pltpu.VMEM((CH_S, D), jnp.float32),
        pltpu.VMEM((CH_S, D), jnp.float32),
        pltpu.VMEM((CH_S, D), jnp.float32),
        pltpu.VMEM((CH_S, D), jnp.float32),
        pltpu.VMEM_SHARED((N_PAD, D), jnp.float32),
        pltpu.SemaphoreType.DMA,
        pltpu.SemaphoreType.DMA,
        pltpu.SemaphoreType.DMA,
        pltpu.SemaphoreType.DMA,
    ],
)
def _scatter_sc(w_hbm, u_hbm, s2d_hbm, r2d_hbm, out_hbm,
                sidx, ridx, w0, u0, w1, u1, acc,
                sem_l0, sem_l1, sem_sc0, sem_sc1):
    cid = lax.axis_index("c")
    sid = lax.axis_index("s")
    wid = sid * _NC + cid

    zero16 = jnp.zeros((16,), jnp.float32)

    def zrow(i, _):
        for q in range(D // 16):
            w0[i, pl.ds(q * 16, 16)] = zero16
        return 0

    lax.fori_loop(0, CH_S, zrow, 0)

    def zcopy(t, _):
        pltpu.sync_copy(w0, acc.at[pl.ds(sid * RPS + t * CH_S, CH_S)])
        return 0

    lax.fori_loop(0, RPS // CH_S, zcopy, 0)
    plsc.subcore_barrier()

    slots = ((w0, u0, sem_l0, sem_sc0), (w1, u1, sem_l1, sem_sc1))

    def half(h, _):
        krow = wid * KCS + h * SEG
        pltpu.sync_copy(s2d_hbm.at[pl.ds(krow, SEG)], sidx)
        pltpu.sync_copy(r2d_hbm.at[pl.ds(krow, SEG)], ridx)
        ebase = wid * EPW_S + h * SEG * CH_S

        def fire_l(c, slot):
            w, u, sl_, _ = slots[slot]
            pltpu.async_copy(w_hbm.at[pl.ds(ebase + c * CH_S, CH_S)], w, sl_)
            pltpu.async_copy(u_hbm.at[sidx.at[c]], u, sl_)

        def wait_l(slot):
            w, u, sl_, _ = slots[slot]
            pltpu.make_async_copy(w_hbm.at[pl.ds(0, CH_S)], w, sl_).wait()
            pltpu.make_async_copy(u_hbm.at[sidx.at[0]], u, sl_).wait()

        def wait_sc(slot):
            w, _, _, ssc = slots[slot]
            pltpu.make_async_copy(w, acc.at[ridx.at[0]], ssc).wait()

        fire_l(0, 0)

        def group(t, _):
            for j in (0, 1):
                c = 2 * t + j
                w, u, _, ssc = slots[j]
                wait_l(j)
                if j == 0:
                    @pl.when(t > 0)
                    def _():
                        wait_sc(1)

                    fire_l(c + 1, 1)
                else:
                    wait_sc(0)

                    @pl.when(t < TG_S - 1)
                    def _():
                        fire_l(c + 1, 0)

                def mrow(i, _):
                    for q in range(D // 16):
                        sl = pl.ds(q * 16, 16)
                        w[i, sl] = w[i, sl] * u[i, sl]
                    return 0

                lax.fori_loop(0, CH_S, mrow, 0)
                pltpu.async_copy(w, acc.at[ridx.at[c]], ssc, add=True)
            return 0

        lax.fori_loop(0, TG_S, group, 0)
        wait_sc(1)
        return 0

    lax.fori_loop(0, NSEG, half, 0)
    plsc.subcore_barrier()

    def wout(t, _):
        rb = sid * RPS + t * CH_S
        pltpu.sync_copy(acc.at[pl.ds(rb, CH_S)],
                        out_hbm.at[pl.ds(cid * N_PAD + rb, CH_S)])
        return 0

    lax.fori_loop(0, RPS // CH_S, wout, 0)


def _mlp_body(g_ref, ef_ref, ea_ref, w1e_ref, b1_ref, w2_ref, b2_ref,
              w3_ref, w_ref):
    g = g_ref[...]
    et = jnp.dot(ef_ref[...], w1e_ref[...], preferred_element_type=jnp.float32)
    h = g + et + b1_ref[...]
    h = h * jax.nn.sigmoid(h)
    h = jnp.dot(h, w2_ref[...], preferred_element_type=jnp.float32) + b2_ref[...]
    h = h * jax.nn.sigmoid(h)
    tpw = jnp.dot(h, w3_ref[...], preferred_element_type=jnp.float32)
    w_ref[...] = tpw * ea_ref[...]


def _mlp(G, ef_ext, edge_attrs, W1e, b1, W2, b2, W3):
    eblk = pl.BlockSpec((EBLK, D), lambda i: (i, 0))
    return pl.pallas_call(
        _mlp_body,
        grid=(G.shape[0] // EBLK,),
        in_specs=[
            eblk,
            pl.BlockSpec((EBLK, 17), lambda i: (i, 0)),
            pl.BlockSpec((EBLK, 1), lambda i: (i, 0)),
            pl.BlockSpec((17, D), lambda i: (0, 0)),
            pl.BlockSpec((1, D), lambda i: (0, 0)),
            pl.BlockSpec((D, D), lambda i: (0, 0)),
            pl.BlockSpec((1, D), lambda i: (0, 0)),
            pl.BlockSpec((D, D), lambda i: (0, 0)),
        ],
        out_specs=eblk,
        out_shape=jax.ShapeDtypeStruct((G.shape[0], D), jnp.float32),
    )(G, ef_ext, edge_attrs, W1e, b1, W2, b2, W3)


def _final_body(m_ref, wout_ref, out_ref):
    m = (m_ref[0] + m_ref[1]) + (m_ref[2] + m_ref[3])
    out_ref[...] = jnp.dot(m, wout_ref[...],
                           preferred_element_type=jnp.float32) * (1.0 / AVG_NUM_NEIGHBORS)


def _final(message_parts, W_out):
    return pl.pallas_call(
        _final_body,
        grid=(N // NBLK,),
        in_specs=[
            pl.BlockSpec((2 * NSEC, NBLK, D), lambda i: (0, i, 0)),
            pl.BlockSpec((D, D), lambda i: (0, 0)),
        ],
        out_specs=pl.BlockSpec((NBLK, D), lambda i: (i, 0)),
        out_shape=jax.ShapeDtypeStruct((N, D), jnp.float32),
    )(message_parts, W_out)


def kernel(node_feats, edge_attrs, edge_feats, lengths, edge_index,
           W_scalar, W_up, W1, b1, W2, b2, W3, W_out):
    sender = edge_index[0].astype(jnp.int32)
    receiver = edge_index[1].astype(jnp.int32)

    W1s = W1[:D]
    W1r = W1[D:2 * D]
    W1e = W1[2 * D:]

    P_s, P_r, U = _precompute(node_feats, W_scalar, W_up, W1s, W1r)

    pad = E_PAD - E
    s2d = jnp.pad(sender, (0, pad)).reshape(ROWS2DS, CH_S)
    r2d = jnp.pad(receiver, (0, pad)).reshape(ROWS2DS, CH_S)
    ef_ext = jnp.pad(jnp.concatenate([edge_feats, lengths], axis=1),
                     ((0, pad), (0, 0)))
    ea_pad = jnp.pad(edge_attrs, ((0, pad), (0, 0)))

    idx_rps = ROWS2DS // NSEC
    e_sec = E_SEC
    parts = []
    ws = []
    for s in range(NSEC):
        s2d_h = lax.slice_in_dim(s2d, s * idx_rps, (s + 1) * idx_rps)
        r2d_h = lax.slice_in_dim(r2d, s * idx_rps, (s + 1) * idx_rps)
        G = _gather_sc(P_s, P_r, s2d_h, r2d_h)
        w = _mlp(G, lax.slice_in_dim(ef_ext, s * e_sec, (s + 1) * e_sec),
                 lax.slice_in_dim(ea_pad, s * e_sec, (s + 1) * e_sec), W1e,
                 b1.reshape(1, D), W2, b2.reshape(1, D), W3)
        ws.append((w, s2d_h, r2d_h))
    for w, s2d_h, r2d_h in ws:
        parts.append(_scatter_sc(w, U, s2d_h, r2d_h).reshape(_NC, N_PAD, D))
    message_parts = jnp.concatenate(parts, axis=0)[:, :N, :]

    out = _final(message_parts, W_out)
    return out.reshape(N, D, 1)


# speedup vs baseline: 1.2765x; 1.0003x over previous
import functools

import jax
import jax.numpy as jnp
from jax import lax
from jax.experimental import pallas as pl
from jax.experimental.pallas import tpu as pltpu
from jax.experimental.pallas import tpu_sc as plsc

N = 10000
E = 320000
D = 128
AVG_NUM_NEIGHBORS = 32.0

NBLK = 1000
EBLK = 2048

_NC, _NS = 2, 16
NW = _NC * _NS
CH = 128
KCH = 80
EPW = KCH * CH
E_PAD = NW * EPW
ROWS2D = E_PAD // CH
N_PAD = 10240
RPS = N_PAD // _NS
ZROWS = 128

_sc_mesh = plsc.VectorSubcoreMesh(core_axis_name="c", subcore_axis_name="s")


def _precompute_body(nf_ref, wsc_ref, wup_ref, w1s_ref, w1r_ref,
                     ps_ref, pr_ref, u_ref):
    nf = nf_ref[...]
    ns = jnp.dot(nf, wsc_ref[...], preferred_element_type=jnp.float32)
    ps_ref[...] = jnp.dot(ns, w1s_ref[...], preferred_element_type=jnp.float32)
    pr_ref[...] = jnp.dot(ns, w1r_ref[...], preferred_element_type=jnp.float32)
    u_ref[...] = jnp.dot(nf, wup_ref[...], preferred_element_type=jnp.float32)


def _precompute(node_feats, W_scalar, W_up, W1s, W1r):
    blk = pl.BlockSpec((NBLK, D), lambda i: (i, 0))
    wblk = pl.BlockSpec((D, D), lambda i: (0, 0))
    return pl.pallas_call(
        _precompute_body,
        grid=(N // NBLK,),
        in_specs=[blk, wblk, wblk, wblk, wblk],
        out_specs=[blk, blk, blk],
        out_shape=[jax.ShapeDtypeStruct((N, D), jnp.float32)] * 3,
    )(node_feats, W_scalar, W_up, W1s, W1r)


NSEC = 4
E_SEC = E_PAD // NSEC
EPW_S = EPW // NSEC
CHG = 64
KCG = EPW_S // CHG
TGG = KCG // 4


@functools.partial(
    pl.kernel,
    out_type=jax.ShapeDtypeStruct((E_SEC, D), jnp.float32),
    mesh=_sc_mesh,
    scratch_types=[
        pltpu.VMEM((KCG, CHG), jnp.int32),
        pltpu.VMEM((KCG, CHG), jnp.int32),
    ] + [pltpu.VMEM((CHG, D), jnp.float32)] * 8
      + [pltpu.SemaphoreType.DMA] * 8,
)
def _gather_sc(ps_hbm, pr_hbm, s2d_hbm, r2d_hbm, g_hbm,
               sidx, ridx, rs0, rr0, rs1, rr1, rs2, rr2, rs3, rr3,
               sg0, sg1, sg2, sg3, sw0, sw1, sw2, sw3):
    wid = lax.axis_index("s") * _NC + lax.axis_index("c")
    krow = wid * KCG
    pltpu.sync_copy(s2d_hbm.at[pl.ds(krow, KCG)], sidx)
    pltpu.sync_copy(r2d_hbm.at[pl.ds(krow, KCG)], ridx)
    ebase = wid * EPW_S

    slots = ((rs0, rr0, sg0, sw0), (rs1, rr1, sg1, sw1),
             (rs2, rr2, sg2, sw2), (rs3, rr3, sg3, sw3))

    def fire_g(c, slot):
        rs, rr, sg, _ = slots[slot]
        pltpu.async_copy(ps_hbm.at[sidx.at[c]], rs, sg)
        pltpu.async_copy(pr_hbm.at[ridx.at[c]], rr, sg)

    def wait_g(slot):
        rs, rr, sg, _ = slots[slot]
        pltpu.make_async_copy(ps_hbm.at[sidx.at[0]], rs, sg).wait()
        pltpu.make_async_copy(pr_hbm.at[ridx.at[0]], rr, sg).wait()

    def wait_wb(slot):
        rs, _, _, swb = slots[slot]
        pltpu.make_async_copy(rs, g_hbm.at[pl.ds(0, CHG)], swb).wait()

    fire_g(0, 0)
    fire_g(1, 1)
    fire_g(2, 2)

    def group(t, _):
        for j in range(4):
            c = 4 * t + j
            rs, rr, _, swb = slots[j]
            nslot = (j + 3) % 4
            wait_g(j)
            if j == 0:
                @pl.when(t > 0)
                def _():
                    wait_wb(nslot)

                fire_g(c + 3, nslot)
            else:
                wait_wb(nslot)

                @pl.when(t < TGG - 1)
                def _():
                    fire_g(c + 3, nslot)

            def add_row(i, _):
                for q in range(D // 16):
                    sl = pl.ds(q * 16, 16)
                    rs[i, sl] = rs[i, sl] + rr[i, sl]
                return 0

            lax.fori_loop(0, CHG, add_row, 0)
            pltpu.async_copy(rs, g_hbm.at[pl.ds(ebase + c * CHG, CHG)], swb)
        return 0

    lax.fori_loop(0, TGG, group, 0)
    wait_wb(3)


CH_S = 64
KCS = EPW_S // CH_S
NSEG = 1
SEG = KCS // NSEG
TG_S = SEG // 2
ROWS2DS = E_PAD // CH_S


@functools.partial(
    pl.kernel,
    out_type=jax.ShapeDtypeStruct((_NC * N_PAD, D), jnp.float32),
    mesh=_sc_mesh,
    scratch_types=[
        pltpu.VMEM((SEG, CH_S), jnp.int32),
        pltpu.VMEM((SEG, CH_S), jnp.int32),
        pltpu.VMEM((CH_S, D), jnp.float32),
        pltpu.VMEM((CH_S, D), jnp.float32),
        pltpu.VMEM((CH_S, D), jnp.float32),
        pltpu.VMEM((CH_S, D), jnp.float32),
        pltpu.VMEM_SHARED((N_PAD, D), jnp.float32),
        pltpu.SemaphoreType.DMA,
        pltpu.SemaphoreType.DMA,
        pltpu.SemaphoreType.DMA,
        pltpu.SemaphoreType.DMA,
    ],
)
def _scatter_sc(w_hbm, u_hbm, s2d_hbm, r2d_hbm, out_hbm,
                sidx, ridx, w0, u0, w1, u1, acc,
                sem_l0, sem_l1, sem_sc0, sem_sc1):
    cid = lax.axis_index("c")
    sid = lax.axis_index("s")
    wid = sid * _NC + cid

    zero16 = jnp.zeros((16,), jnp.float32)

    def zrow(i, _):
        for q in range(D // 16):
            w0[i, pl.ds(q * 16, 16)] = zero16
        return 0

    lax.fori_loop(0, CH_S, zrow, 0)

    def zcopy(t, _):
        pltpu.sync_copy(w0, acc.at[pl.ds(sid * RPS + t * CH_S, CH_S)])
        return 0

    lax.fori_loop(0, RPS // CH_S, zcopy, 0)
    plsc.subcore_barrier()

    slots = ((w0, u0, sem_l0, sem_sc0), (w1, u1, sem_l1, sem_sc1))

    def half(h, _):
        krow = wid * KCS + h * SEG
        pltpu.sync_copy(s2d_hbm.at[pl.ds(krow, SEG)], sidx)
        pltpu.sync_copy(r2d_hbm.at[pl.ds(krow, SEG)], ridx)
        ebase = wid * EPW_S + h * SEG * CH_S

        def fire_l(c, slot):
            w, u, sl_, _ = slots[slot]
            pltpu.async_copy(w_hbm.at[pl.ds(ebase + c * CH_S, CH_S)], w, sl_)
            pltpu.async_copy(u_hbm.at[sidx.at[c]], u, sl_)

        def wait_l(slot):
            w, u, sl_, _ = slots[slot]
            pltpu.make_async_copy(w_hbm.at[pl.ds(0, CH_S)], w, sl_).wait()
            pltpu.make_async_copy(u_hbm.at[sidx.at[0]], u, sl_).wait()

        def wait_sc(slot):
            w, _, _, ssc = slots[slot]
            pltpu.make_async_copy(w, acc.at[ridx.at[0]], ssc).wait()

        fire_l(0, 0)

        def group(t, _):
            for j in (0, 1):
                c = 2 * t + j
                w, u, _, ssc = slots[j]
                wait_l(j)
                if j == 0:
                    @pl.when(t > 0)
                    def _():
                        wait_sc(1)

                    fire_l(c + 1, 1)
                else:
                    wait_sc(0)

                    @pl.when(t < TG_S - 1)
                    def _():
                        fire_l(c + 1, 0)

                def mrow(i, _):
                    for q in range(D // 16):
                        sl = pl.ds(q * 16, 16)
                        w[i, sl] = w[i, sl] * u[i, sl]
                    return 0

                lax.fori_loop(0, CH_S, mrow, 0)
                pltpu.async_copy(w, acc.at[ridx.at[c]], ssc, add=True)
            return 0

        lax.fori_loop(0, TG_S, group, 0)
        wait_sc(1)
        return 0

    lax.fori_loop(0, NSEG, half, 0)
    plsc.subcore_barrier()

    def wout(t, _):
        rb = sid * RPS + t * CH_S
        pltpu.sync_copy(acc.at[pl.ds(rb, CH_S)],
                        out_hbm.at[pl.ds(cid * N_PAD + rb, CH_S)])
        return 0

    lax.fori_loop(0, RPS // CH_S, wout, 0)


def _mlp_body(g_ref, ef_ref, ea_ref, w1e_ref, b1_ref, w2_ref, b2_ref,
              w3_ref, w_ref):
    g = g_ref[...]
    et = jnp.dot(ef_ref[...], w1e_ref[...], preferred_element_type=jnp.float32)
    h = g + et + b1_ref[...]
    h = h * jax.nn.sigmoid(h)
    h = jnp.dot(h, w2_ref[...], preferred_element_type=jnp.float32) + b2_ref[...]
    h = h * jax.nn.sigmoid(h)
    tpw = jnp.dot(h, w3_ref[...], preferred_element_type=jnp.float32)
    w_ref[...] = tpw * ea_ref[...]


def _mlp(G, ef_ext, edge_attrs, W1e, b1, W2, b2, W3):
    eblk = pl.BlockSpec((EBLK, D), lambda i: (i, 0))
    return pl.pallas_call(
        _mlp_body,
        grid=(G.shape[0] // EBLK,),
        in_specs=[
            eblk,
            pl.BlockSpec((EBLK, 17), lambda i: (i, 0)),
            pl.BlockSpec((EBLK, 1), lambda i: (i, 0)),
            pl.BlockSpec((17, D), lambda i: (0, 0)),
            pl.BlockSpec((1, D), lambda i: (0, 0)),
            pl.BlockSpec((D, D), lambda i: (0, 0)),
            pl.BlockSpec((1, D), lambda i: (0, 0)),
            pl.BlockSpec((D, D), lambda i: (0, 0)),
        ],
        out_specs=eblk,
        out_shape=jax.ShapeDtypeStruct((G.shape[0], D), jnp.float32),
    )(G, ef_ext, edge_attrs, W1e, b1, W2, b2, W3)


def _final_body(m_ref, wout_ref, out_ref):
    parts = [m_ref[i] for i in range(2 * NSEC)]
    while len(parts) > 1:
        parts = [a + b for a, b in zip(parts[::2], parts[1::2])]
    m = parts[0]
    out_ref[...] = jnp.dot(m, wout_ref[...],
                           preferred_element_type=jnp.float32) * (1.0 / AVG_NUM_NEIGHBORS)


def _final(message_parts, W_out):
    return pl.pallas_call(
        _final_body,
        grid=(N // NBLK,),
        in_specs=[
            pl.BlockSpec((2 * NSEC, NBLK, D), lambda i: (0, i, 0)),
            pl.BlockSpec((D, D), lambda i: (0, 0)),
        ],
        out_specs=pl.BlockSpec((NBLK, D), lambda i: (i, 0)),
        out_shape=jax.ShapeDtypeStruct((N, D), jnp.float32),
    )(message_parts, W_out)


def kernel(node_feats, edge_attrs, edge_feats, lengths, edge_index,
           W_scalar, W_up, W1, b1, W2, b2, W3, W_out):
    sender = edge_index[0].astype(jnp.int32)
    receiver = edge_index[1].astype(jnp.int32)

    W1s = W1[:D]
    W1r = W1[D:2 * D]
    W1e = W1[2 * D:]

    P_s, P_r, U = _precompute(node_feats, W_scalar, W_up, W1s, W1r)

    pad = E_PAD - E
    s2d = jnp.pad(sender, (0, pad)).reshape(ROWS2DS, CH_S)
    r2d = jnp.pad(receiver, (0, pad)).reshape(ROWS2DS, CH_S)
    ef_ext = jnp.pad(jnp.concatenate([edge_feats, lengths], axis=1),
                     ((0, pad), (0, 0)))
    ea_pad = jnp.pad(edge_attrs, ((0, pad), (0, 0)))

    idx_rps = ROWS2DS // NSEC
    e_sec = E_SEC
    parts = []
    ws = []
    for s in range(NSEC):
        s2d_h = lax.slice_in_dim(s2d, s * idx_rps, (s + 1) * idx_rps)
        r2d_h = lax.slice_in_dim(r2d, s * idx_rps, (s + 1) * idx_rps)
        G = _gather_sc(P_s, P_r, s2d_h, r2d_h)
        w = _mlp(G, lax.slice_in_dim(ef_ext, s * e_sec, (s + 1) * e_sec),
                 lax.slice_in_dim(ea_pad, s * e_sec, (s + 1) * e_sec), W1e,
                 b1.reshape(1, D), W2, b2.reshape(1, D), W3)
        ws.append((w, s2d_h, r2d_h))
    for w, s2d_h, r2d_h in ws:
        parts.append(_scatter_sc(w, U, s2d_h, r2d_h).reshape(_NC, N_PAD, D))
    message_parts = jnp.concatenate(parts, axis=0)[:, :N, :]

    out = _final(message_parts, W_out)
    return out.reshape(N, D, 1)
